# baseline (speedup 1.0000x reference)
import functools

import jax
import jax.numpy as jnp
from jax import lax
from jax.experimental import pallas as pl
from jax.experimental.pallas import tpu as pltpu


def _ln_rows_kernel(x_ref, w_ref, b_ref, o_ref, *, eps, inv_c):
    x = x_ref[...]
    s = jnp.sum(x, axis=-1, keepdims=True)
    sq = jnp.sum(x * x, axis=-1, keepdims=True)
    mu = s * inv_c
    var = sq * inv_c - mu * mu
    inv = lax.rsqrt(var + eps)
    o_ref[...] = (x - mu) * inv * w_ref[...] + b_ref[...]


def kernel(x, weight, bias, *, eps=1e-6):
    c = x.shape[-1]
    lead = x.shape[:-1]
    x2d = x.reshape(-1, c)
    rows = x2d.shape[0]

    tile = 512
    if rows < tile:
        tile = rows

    kernel_fn = functools.partial(_ln_rows_kernel, eps=eps, inv_c=1.0 / c)
    y2d = pl.pallas_call(
        kernel_fn,
        out_shape=jax.ShapeDtypeStruct((rows, c), x.dtype),
        grid=(pl.cdiv(rows, tile),),
        in_specs=[
            pl.BlockSpec((tile, c), lambda i: (i, 0)),
            pl.BlockSpec((1, c), lambda i: (0, 0)),
            pl.BlockSpec((1, c), lambda i: (0, 0)),
        ],
        out_specs=pl.BlockSpec((tile, c), lambda i: (i, 0)),
        compiler_params=pltpu.CompilerParams(
            dimension_semantics=("parallel",),
            vmem_limit_bytes=32 * 1024 * 1024,
        ),
    )(x2d, weight.reshape(1, c), bias.reshape(1, c))
    return y2d.reshape(*lead, c)


# speedup vs baseline: 1.1368x; 1.1368x over previous
import functools

import jax
import jax.numpy as jnp
from jax import lax
from jax.experimental import pallas as pl
from jax.experimental.pallas import tpu as pltpu


def _ln_rows_kernel(x_ref, w_ref, b_ref, o_ref, *, eps, inv_c):
    x = x_ref[...]
    s = jnp.sum(x, axis=-1, keepdims=True)
    sq = jnp.sum(x * x, axis=-1, keepdims=True)
    mu = s * inv_c
    var = sq * inv_c - mu * mu
    inv = lax.rsqrt(var + eps)
    o_ref[...] = (x - mu) * inv * w_ref[...] + b_ref[...]


def kernel(x, weight, bias, *, eps=1e-6):
    c = x.shape[-1]
    lead = x.shape[:-1]
    x2d = x.reshape(-1, c)
    rows = x2d.shape[0]

    tile = 1024
    if rows < tile:
        tile = rows

    kernel_fn = functools.partial(_ln_rows_kernel, eps=eps, inv_c=1.0 / c)
    y2d = pl.pallas_call(
        kernel_fn,
        out_shape=jax.ShapeDtypeStruct((rows, c), x.dtype),
        grid=(pl.cdiv(rows, tile),),
        in_specs=[
            pl.BlockSpec((tile, c), lambda i: (i, 0)),
            pl.BlockSpec((1, c), lambda i: (0, 0)),
            pl.BlockSpec((1, c), lambda i: (0, 0)),
        ],
        out_specs=pl.BlockSpec((tile, c), lambda i: (i, 0)),
        compiler_params=pltpu.CompilerParams(
            dimension_semantics=("parallel",),
            vmem_limit_bytes=32 * 1024 * 1024,
        ),
    )(x2d, weight.reshape(1, c), bias.reshape(1, c))
    return y2d.reshape(*lead, c)


# speedup vs baseline: 1.1631x; 1.0231x over previous
import functools

import jax
import jax.numpy as jnp
from jax import lax
from jax.experimental import pallas as pl
from jax.experimental.pallas import tpu as pltpu


def _ln_rows_kernel(x_ref, w_ref, b_ref, o_ref, *, eps, inv_c):
    x = x_ref[...]
    s = jnp.sum(x, axis=-1, keepdims=True)
    sq = jnp.sum(x * x, axis=-1, keepdims=True)
    mu = s * inv_c
    var = sq * inv_c - mu * mu
    inv = lax.rsqrt(var + eps)
    o_ref[...] = (x - mu) * inv * w_ref[...] + b_ref[...]


def kernel(x, weight, bias, *, eps=1e-6):
    c = x.shape[-1]
    lead = x.shape[:-1]
    x2d = x.reshape(-1, c)
    rows = x2d.shape[0]

    tile = 2048
    if rows < tile:
        tile = rows

    kernel_fn = functools.partial(_ln_rows_kernel, eps=eps, inv_c=1.0 / c)
    y2d = pl.pallas_call(
        kernel_fn,
        out_shape=jax.ShapeDtypeStruct((rows, c), x.dtype),
        grid=(pl.cdiv(rows, tile),),
        in_specs=[
            pl.BlockSpec((tile, c), lambda i: (i, 0)),
            pl.BlockSpec((1, c), lambda i: (0, 0)),
            pl.BlockSpec((1, c), lambda i: (0, 0)),
        ],
        out_specs=pl.BlockSpec((tile, c), lambda i: (i, 0)),
        compiler_params=pltpu.CompilerParams(
            dimension_semantics=("parallel",),
            vmem_limit_bytes=48 * 1024 * 1024,
        ),
    )(x2d, weight.reshape(1, c), bias.reshape(1, c))
    return y2d.reshape(*lead, c)
